# Initial kernel scaffold; baseline (speedup 1.0000x reference)
import jax
import jax.numpy as jnp
from jax.experimental import pallas as pl


def kernel(x, emb_weight):
    raise NotImplementedError("write your pallas kernel here")


# speedup vs baseline: 1.0633x; 1.0633x over previous
import functools

import jax
import jax.numpy as jnp
from jax import lax
from jax.experimental import pallas as pl
from jax.experimental.pallas import tpu as pltpu
from jax.experimental.pallas import tpu_sc as plsc

N_ROWS = 16384
N_CODES = 8192
DIM = 256
BLOCK_ROWS = 256
N_BLOCKS = N_ROWS // BLOCK_ROWS


def _distance_argmin_kernel(x_ref, emb_ref, idx_ref, dsum_ref, e2_ref, acc_ref):
    i = pl.program_id(0)
    emb = emb_ref[...]

    @pl.when(i == 0)
    def _init():
        e2_ref[0, :] = jnp.sum(emb * emb, axis=1)
        acc_ref[0, 0] = 0.0

    x_blk = x_ref[...]
    m = lax.dot_general(x_blk, emb, (((1,), (1,)), ((), ())),
                        preferred_element_type=jnp.float32)
    x2 = jnp.sum(x_blk * x_blk, axis=1, keepdims=True)
    scores = (x2 + e2_ref[0, :][None, :]) - 2.0 * m
    idx_ref[...] = jnp.argmin(scores, axis=1).astype(jnp.int32)
    dmin = jnp.min(scores, axis=1)
    acc_ref[0, 0] += jnp.sum(dmin)

    @pl.when(i == pl.num_programs(0) - 1)
    def _fin():
        dsum_ref[0, 0] = acc_ref[0, 0]


def _nearest_codes(x, emb_weight):
    return pl.pallas_call(
        _distance_argmin_kernel,
        grid=(N_BLOCKS,),
        in_specs=[
            pl.BlockSpec((BLOCK_ROWS, DIM), lambda i: (i, 0)),
            pl.BlockSpec((N_CODES, DIM), lambda i: (0, 0)),
        ],
        out_specs=[
            pl.BlockSpec((BLOCK_ROWS,), lambda i: (i,)),
            pl.BlockSpec(memory_space=pltpu.SMEM),
        ],
        out_shape=[
            jax.ShapeDtypeStruct((N_ROWS,), jnp.int32),
            jax.ShapeDtypeStruct((1, 1), jnp.float32),
        ],
        scratch_shapes=[
            pltpu.VMEM((1, N_CODES), jnp.float32),
            pltpu.SMEM((1, 1), jnp.float32),
        ],
    )(x, emb_weight)


_SC_CHUNK = 128


def _sc_gather_kernel(emb_hbm, idx_hbm, out_hbm, idx_v, rows_v, sem):
    n_cores = 2
    wid = lax.axis_index("s") * n_cores + lax.axis_index("c")
    rows_per_w = N_ROWS // 32
    for c in range(rows_per_w // _SC_CHUNK):
        base = wid * rows_per_w + c * _SC_CHUNK
        pltpu.sync_copy(idx_hbm.at[pl.ds(base, _SC_CHUNK)], idx_v)
        pltpu.async_copy(emb_hbm.at[idx_v], rows_v, sem).wait()
        pltpu.sync_copy(rows_v, out_hbm.at[pl.ds(base, _SC_CHUNK)])


def _gather_rows(emb_weight, idx):
    mesh = plsc.VectorSubcoreMesh(core_axis_name="c", subcore_axis_name="s")
    k = functools.partial(
        pl.kernel,
        out_type=jax.ShapeDtypeStruct((N_ROWS, DIM), jnp.float32),
        mesh=mesh,
        scratch_types=[
            pltpu.VMEM((_SC_CHUNK,), jnp.int32),
            pltpu.VMEM((_SC_CHUNK, DIM), jnp.float32),
            pltpu.SemaphoreType.DMA,
        ],
    )(_sc_gather_kernel)
    return k(emb_weight, idx)


def kernel(x, emb_weight):
    idx, dsum = _nearest_codes(x, emb_weight)
    quantized = _gather_rows(emb_weight, idx)
    loss = dsum[0, 0] * (1.25 / (N_ROWS * DIM))
    return (quantized, loss)


# speedup vs baseline: 1.3889x; 1.3062x over previous
import functools

import jax
import jax.numpy as jnp
from jax import lax
from jax.experimental import pallas as pl
from jax.experimental.pallas import tpu as pltpu
from jax.experimental.pallas import tpu_sc as plsc

N_ROWS = 16384
N_CODES = 8192
DIM = 256
BLOCK_ROWS = 256
N_BLOCKS = N_ROWS // BLOCK_ROWS


CHUNK = 256
N_CHUNKS = N_CODES // CHUNK


def _distance_argmin_kernel(x_ref, emb_ref, idx_ref, dsum_ref,
                            e2_ref, embt_ref, acc_ref):
    i = pl.program_id(0)

    @pl.when(i == 0)
    def _init():
        emb = emb_ref[...]
        e2_ref[0, :] = jnp.sum(emb * emb, axis=1)
        embt_ref[...] = emb.T.astype(jnp.bfloat16)
        acc_ref[0, 0] = 0.0

    x_blk = x_ref[...]
    x2 = jnp.sum(x_blk * x_blk, axis=1, keepdims=True)
    xs = (-2.0 * x_blk).astype(jnp.bfloat16)

    lane = lax.broadcasted_iota(jnp.int32, (BLOCK_ROWS, 128), 1)
    val = jnp.full((BLOCK_ROWS, 128), jnp.inf, jnp.float32)
    cidx = jnp.zeros((BLOCK_ROWS, 128), jnp.int32)
    for j in range(N_CHUNKS):
        m = jnp.dot(xs, embt_ref[:, j * CHUNK:(j + 1) * CHUNK],
                    preferred_element_type=jnp.float32)
        e2c = e2_ref[0, j * CHUNK:(j + 1) * CHUNK]
        s = (x2 + e2c[None, :]) + m
        s0 = s[:, :128]
        s1 = s[:, 128:]
        c0 = lane + j * CHUNK
        h1 = s1 < s0
        sc = jnp.where(h1, s1, s0)
        cc = jnp.where(h1, c0 + 128, c0)
        better = sc < val
        cidx = jnp.where(better, cc, cidx)
        val = jnp.where(better, sc, val)

    minv = jnp.min(val, axis=1)
    sel = jnp.where(val == minv[:, None], cidx, jnp.int32(N_CODES))
    idx_ref[...] = jnp.min(sel, axis=1)
    acc_ref[0, 0] += jnp.sum(minv)

    @pl.when(i == pl.num_programs(0) - 1)
    def _fin():
        dsum_ref[0, 0] = acc_ref[0, 0]


def _nearest_codes(x, emb_weight):
    return pl.pallas_call(
        _distance_argmin_kernel,
        grid=(N_BLOCKS,),
        in_specs=[
            pl.BlockSpec((BLOCK_ROWS, DIM), lambda i: (i, 0)),
            pl.BlockSpec((N_CODES, DIM), lambda i: (0, 0)),
        ],
        out_specs=[
            pl.BlockSpec((BLOCK_ROWS,), lambda i: (i,)),
            pl.BlockSpec(memory_space=pltpu.SMEM),
        ],
        out_shape=[
            jax.ShapeDtypeStruct((N_ROWS,), jnp.int32),
            jax.ShapeDtypeStruct((1, 1), jnp.float32),
        ],
        scratch_shapes=[
            pltpu.VMEM((1, N_CODES), jnp.float32),
            pltpu.VMEM((DIM, N_CODES), jnp.bfloat16),
            pltpu.SMEM((1, 1), jnp.float32),
        ],
    )(x, emb_weight)


_SC_CHUNK = 128


def _sc_gather_kernel(emb_hbm, idx_hbm, out_hbm, idx_v, rows_v, sem):
    n_cores = 2
    wid = lax.axis_index("s") * n_cores + lax.axis_index("c")
    rows_per_w = N_ROWS // 32
    for c in range(rows_per_w // _SC_CHUNK):
        base = wid * rows_per_w + c * _SC_CHUNK
        pltpu.sync_copy(idx_hbm.at[pl.ds(base, _SC_CHUNK)], idx_v)
        pltpu.async_copy(emb_hbm.at[idx_v], rows_v, sem).wait()
        pltpu.sync_copy(rows_v, out_hbm.at[pl.ds(base, _SC_CHUNK)])


def _gather_rows(emb_weight, idx):
    mesh = plsc.VectorSubcoreMesh(core_axis_name="c", subcore_axis_name="s")
    k = functools.partial(
        pl.kernel,
        out_type=jax.ShapeDtypeStruct((N_ROWS, DIM), jnp.float32),
        mesh=mesh,
        scratch_types=[
            pltpu.VMEM((_SC_CHUNK,), jnp.int32),
            pltpu.VMEM((_SC_CHUNK, DIM), jnp.float32),
            pltpu.SemaphoreType.DMA,
        ],
    )(_sc_gather_kernel)
    return k(emb_weight, idx)


def kernel(x, emb_weight):
    idx, dsum = _nearest_codes(x, emb_weight)
    quantized = _gather_rows(emb_weight, idx)
    loss = dsum[0, 0] * (1.25 / (N_ROWS * DIM))
    return (quantized, loss)
